# baseline (speedup 1.0000x reference)
import jax
import jax.numpy as jnp
from jax.experimental import pallas as pl
from jax.experimental.pallas import tpu as pltpu

N = 4096
D_IN = 64
D_OUT = 64
BM = 256
NSTEPS = N // BM


def _gcn_kernel(
    infeatn_hbm, adj_ref, w_ref, b_ref, out_hbm,
    support_ref, infeatn_vmem, out_stage,
    in_sem, out_sems,
):
    step = pl.program_id(0)

    @pl.when(step == 0)
    def _():
        copy = pltpu.make_async_copy(infeatn_hbm, infeatn_vmem, in_sem)
        copy.start()
        copy.wait()
        support_ref[...] = jnp.dot(
            infeatn_vmem[...], w_ref[...], preferred_element_type=jnp.float32
        )

    slot = jax.lax.rem(step, 2)

    @pl.when(step >= 2)
    def _():
        pltpu.make_async_copy(
            out_stage.at[slot],
            out_hbm.at[pl.ds((step - 2) * BM, BM), :],
            out_sems.at[slot],
        ).wait()

    out_stage[slot] = (
        jnp.dot(adj_ref[...], support_ref[...], preferred_element_type=jnp.float32)
        + b_ref[...]
    )
    pltpu.make_async_copy(
        out_stage.at[slot],
        out_hbm.at[pl.ds(step * BM, BM), :],
        out_sems.at[slot],
    ).start()

    @pl.when(step == NSTEPS - 1)
    def _():
        for s, st in ((0, NSTEPS - 2), (1, NSTEPS - 1)):
            pltpu.make_async_copy(
                out_stage.at[s],
                out_hbm.at[pl.ds(st * BM, BM), :],
                out_sems.at[s],
            ).wait()


@jax.jit
def kernel(infeatn, adj, W, b):
    b2 = b.reshape(1, D_OUT)
    grid = (NSTEPS,)
    return pl.pallas_call(
        _gcn_kernel,
        grid=grid,
        in_specs=[
            pl.BlockSpec(memory_space=pl.ANY),
            pl.BlockSpec((BM, N), lambda i: (i, 0)),
            pl.BlockSpec((D_IN, D_OUT), lambda i: (0, 0)),
            pl.BlockSpec((1, D_OUT), lambda i: (0, 0)),
        ],
        out_specs=pl.BlockSpec(memory_space=pl.ANY),
        out_shape=jax.ShapeDtypeStruct((N, D_OUT), jnp.float32),
        scratch_shapes=[
            pltpu.VMEM((N, D_OUT), jnp.float32),
            pltpu.VMEM((N, D_IN), jnp.float32),
            pltpu.VMEM((2, BM, D_OUT), jnp.float32),
            pltpu.SemaphoreType.DMA,
            pltpu.SemaphoreType.DMA((2,)),
        ],
    )(infeatn, adj, W, b2)


# speedup vs baseline: 1.1545x; 1.1545x over previous
import jax
import jax.numpy as jnp
from jax.experimental import pallas as pl
from jax.experimental.pallas import tpu as pltpu

N = 4096
D_IN = 64
D_OUT = 64
BM = 512
NSTEPS = N // BM


def _gcn_kernel(
    infeatn_ref, adj_ref, w_ref, b_ref, out_hbm,
    support_ref, out_stage, out_sems,
):
    step = pl.program_id(0)

    @pl.when(step == 0)
    def _():
        support_ref[...] = jnp.dot(
            infeatn_ref[...], w_ref[...], preferred_element_type=jnp.float32
        )

    slot = jax.lax.rem(step, 2)

    @pl.when(step >= 2)
    def _():
        pltpu.make_async_copy(
            out_stage.at[slot],
            out_hbm.at[pl.ds((step - 2) * BM, BM), :],
            out_sems.at[slot],
        ).wait()

    out_stage[slot] = (
        jnp.dot(adj_ref[...], support_ref[...], preferred_element_type=jnp.float32)
        + b_ref[...]
    )
    pltpu.make_async_copy(
        out_stage.at[slot],
        out_hbm.at[pl.ds(step * BM, BM), :],
        out_sems.at[slot],
    ).start()

    @pl.when(step == NSTEPS - 1)
    def _():
        for s, st in ((0, NSTEPS - 2), (1, NSTEPS - 1)):
            pltpu.make_async_copy(
                out_stage.at[s],
                out_hbm.at[pl.ds(st * BM, BM), :],
                out_sems.at[s],
            ).wait()


@jax.jit
def kernel(infeatn, adj, W, b):
    b2 = b.reshape(1, D_OUT)
    grid = (NSTEPS,)
    return pl.pallas_call(
        _gcn_kernel,
        grid=grid,
        in_specs=[
            pl.BlockSpec((N, D_IN), lambda i: (0, 0)),
            pl.BlockSpec((BM, N), lambda i: (i, 0)),
            pl.BlockSpec((D_IN, D_OUT), lambda i: (0, 0)),
            pl.BlockSpec((1, D_OUT), lambda i: (0, 0)),
        ],
        out_specs=pl.BlockSpec(memory_space=pl.ANY),
        out_shape=jax.ShapeDtypeStruct((N, D_OUT), jnp.float32),
        scratch_shapes=[
            pltpu.VMEM((N, D_OUT), jnp.float32),
            pltpu.VMEM((2, BM, D_OUT), jnp.float32),
            pltpu.SemaphoreType.DMA((2,)),
        ],
    )(infeatn, adj, W, b2)
